# baseline (device time: 222724 ns/iter reference)
import numpy as np
import jax
import jax.numpy as jnp
from jax import lax
from jax.experimental import pallas as pl
from jax.experimental.pallas import tpu as pltpu

N_DEV = 4
SEQ = 1024
D = 1024
HQ = 8
DH = 128
QT = 512
HALF = SEQ // 2
SCALE = 0.08838834764831843

_INV = (1.0 / (10000.0 ** (np.arange(0, DH, 2) / DH))).astype(np.float32)


def kernel(x, Wq, Wk, Wv, Wo):
    xs = x.reshape(SEQ, D)

    my = lax.axis_index("i")
    pos = my * SEQ + jnp.arange(SEQ, dtype=jnp.float32)
    ang = pos[:, None] * jnp.asarray(_INV)[None, :]
    cos = jnp.repeat(jnp.cos(ang), 2, axis=-1)
    sin = jnp.repeat(jnp.sin(ang), 2, axis=-1)

    def rope(t):
        t2 = t.reshape(SEQ, HQ, DH // 2, 2)
        tr = jnp.stack([-t2[..., 1], t2[..., 0]], axis=-1).reshape(SEQ, HQ, DH)
        return t * cos[:, None, :] + tr * sin[:, None, :]

    qkv = xs @ jnp.concatenate([Wq, Wk, Wv], axis=1)
    q = (rope(qkv[:, :D].reshape(SEQ, HQ, DH)) * SCALE).transpose(1, 0, 2)
    k = rope(qkv[:, D:2 * D].reshape(SEQ, HQ, DH)).transpose(1, 0, 2)
    v = qkv[:, 2 * D:].reshape(SEQ, HQ, DH).transpose(1, 0, 2)
    kv = jnp.stack([k, v])
    wo = Wo.reshape(HQ, DH, D)

    def body(q_ref, kv_ref, wo_ref, out_ref,
             kvcom, acc_ref, den_ref, send, recv):
        my_pos = lax.axis_index("i")
        left = lax.rem(my_pos + N_DEV - 1, N_DEV)
        right = lax.rem(my_pos + 1, N_DEV)

        barrier_sem = pltpu.get_barrier_semaphore()
        pl.semaphore_signal(barrier_sem, inc=1, device_id=(left,),
                            device_id_type=pl.DeviceIdType.MESH)
        pl.semaphore_signal(barrier_sem, inc=1, device_id=(right,),
                            device_id_type=pl.DeviceIdType.MESH)
        pl.semaphore_wait(barrier_sem, 2)

        def accumulate(src, init, koff, klen):
            def head_step(h, carry):
                kc = src[0, h, pl.ds(koff, klen)]
                vc = src[1, h, pl.ds(koff, klen)]
                for c in range(SEQ // QT):
                    rows = pl.ds(c * QT, QT)
                    e = jnp.exp(lax.dot_general(
                        q_ref[h, rows], kc, (((1,), (1,)), ((), ())),
                        preferred_element_type=jnp.float32))
                    num = jnp.dot(e, vc, preferred_element_type=jnp.float32)
                    den = jnp.sum(e, axis=-1, keepdims=True)
                    if init:
                        acc_ref[h, rows] = num
                        den_ref[h, rows] = den
                    else:
                        acc_ref[h, rows] = acc_ref[h, rows] + num
                        den_ref[h, rows] = den_ref[h, rows] + den
                return carry

            lax.fori_loop(0, HQ, head_step, 0)

        def remote_copy(src, dst, sem_idx, target):
            return pltpu.make_async_remote_copy(
                src_ref=src, dst_ref=dst,
                send_sem=send.at[sem_idx], recv_sem=recv.at[sem_idx],
                device_id=(target,), device_id_type=pl.DeviceIdType.MESH)

        top = pl.ds(0, HALF)
        bot = pl.ds(HALF, HALF)

        p1r_t = remote_copy(kv_ref.at[:, :, top], kvcom.at[0, :, :, top],
                            0, right)
        p1r_b = remote_copy(kv_ref.at[:, :, bot], kvcom.at[0, :, :, bot],
                            1, right)
        p1l_b = remote_copy(kv_ref.at[:, :, bot], kvcom.at[1, :, :, bot],
                            2, left)
        p1l_t = remote_copy(kv_ref.at[:, :, top], kvcom.at[1, :, :, top],
                            3, left)
        p1r_t.start()
        p1l_b.start()
        p1r_b.start()
        p1l_t.start()

        accumulate(kv_ref, True, 0, SEQ)

        p1r_t.wait_recv()
        p1l_b.wait_recv()

        p2r = remote_copy(kvcom.at[0, :, :, top], kvcom.at[2, :, :, top],
                          4, right)
        p2l = remote_copy(kvcom.at[1, :, :, bot], kvcom.at[2, :, :, bot],
                          5, left)
        p2r.start()
        p2l.start()

        accumulate(kvcom.at[0], False, 0, HALF)
        accumulate(kvcom.at[1], False, HALF, HALF)

        p1r_b.wait_recv()
        p1l_t.wait_recv()

        accumulate(kvcom.at[0], False, HALF, HALF)
        accumulate(kvcom.at[1], False, 0, HALF)

        p2r.wait_recv()
        p2l.wait_recv()

        accumulate(kvcom.at[2], False, 0, SEQ)

        for r in (p1r_t, p1r_b, p1l_b, p1l_t, p2r, p2l):
            r.wait_send()

        out_ref[...] = jnp.zeros((SEQ, D), jnp.float32)

        def proj_step(h, carry):
            ctx_h = acc_ref[h] / den_ref[h]
            out_ref[...] = out_ref[...] + jnp.dot(
                ctx_h, wo_ref[h], preferred_element_type=jnp.float32)
            return carry

        lax.fori_loop(0, HQ, proj_step, 0)

    out2d = pl.pallas_call(
        body,
        out_shape=jax.ShapeDtypeStruct((SEQ, D), jnp.float32),
        in_specs=[pl.BlockSpec(memory_space=pltpu.VMEM)] * 3,
        out_specs=pl.BlockSpec(memory_space=pltpu.VMEM),
        scratch_shapes=[
            pltpu.VMEM((3, 2, HQ, SEQ, DH), jnp.float32),
            pltpu.VMEM((HQ, SEQ, DH), jnp.float32),
            pltpu.VMEM((HQ, SEQ, 1), jnp.float32),
            pltpu.SemaphoreType.DMA((6,)),
            pltpu.SemaphoreType.DMA((6,)),
        ],
        compiler_params=pltpu.CompilerParams(
            collective_id=0, vmem_limit_bytes=100 * 1024 * 1024),
    )(q, kv, wo)

    return out2d.reshape(1, SEQ, D)


# device time: 151168 ns/iter; 1.4734x vs baseline; 1.4734x over previous
import numpy as np
import jax
import jax.numpy as jnp
from jax import lax
from jax.experimental import pallas as pl
from jax.experimental.pallas import tpu as pltpu

N_DEV = 4
SEQ = 1024
D = 1024
HQ = 8
DH = 128
QT = 512
HALF = SEQ // 2
SCALE = 0.08838834764831843

_INV = (1.0 / (10000.0 ** (np.arange(0, DH, 2) / DH))).astype(np.float32)


def kernel(x, Wq, Wk, Wv, Wo):
    xs = x.reshape(SEQ, D)

    my = lax.axis_index("i")
    pos = my * SEQ + jnp.arange(SEQ, dtype=jnp.float32)
    ang = pos[:, None] * jnp.asarray(_INV)[None, :]
    cos = jnp.repeat(jnp.cos(ang), 2, axis=-1)
    sin = jnp.repeat(jnp.sin(ang), 2, axis=-1)

    def rope(t):
        t2 = t.reshape(SEQ, HQ, DH // 2, 2)
        tr = jnp.stack([-t2[..., 1], t2[..., 0]], axis=-1).reshape(SEQ, HQ, DH)
        return t * cos[:, None, :] + tr * sin[:, None, :]

    qkv = xs @ jnp.concatenate([Wq, Wk, Wv], axis=1)
    q = (rope(qkv[:, :D].reshape(SEQ, HQ, DH)) * SCALE).transpose(1, 0, 2)
    k = rope(qkv[:, D:2 * D].reshape(SEQ, HQ, DH)).transpose(1, 0, 2)
    v = qkv[:, 2 * D:].reshape(SEQ, HQ, DH).transpose(1, 0, 2)
    q = q.astype(jnp.bfloat16)
    kv = jnp.stack([k, v]).astype(jnp.bfloat16)
    wo = Wo.reshape(HQ, DH, D)

    def body(q_ref, kv_ref, wo_ref, out_ref,
             kvcom, acc_ref, den_ref, send, recv):
        my_pos = lax.axis_index("i")
        left = lax.rem(my_pos + N_DEV - 1, N_DEV)
        right = lax.rem(my_pos + 1, N_DEV)

        barrier_sem = pltpu.get_barrier_semaphore()
        pl.semaphore_signal(barrier_sem, inc=1, device_id=(left,),
                            device_id_type=pl.DeviceIdType.MESH)
        pl.semaphore_signal(barrier_sem, inc=1, device_id=(right,),
                            device_id_type=pl.DeviceIdType.MESH)
        pl.semaphore_wait(barrier_sem, 2)

        def accumulate(src, init, koff, klen):
            def head_step(h, carry):
                kc = src[0, h, pl.ds(koff, klen)]
                vc = src[1, h, pl.ds(koff, klen)]
                for c in range(SEQ // QT):
                    rows = pl.ds(c * QT, QT)
                    e = jnp.exp(lax.dot_general(
                        q_ref[h, rows], kc, (((1,), (1,)), ((), ())),
                        preferred_element_type=jnp.float32))
                    num = jnp.dot(e.astype(jnp.bfloat16), vc,
                                  preferred_element_type=jnp.float32)
                    den = jnp.sum(e, axis=-1, keepdims=True)
                    if init:
                        acc_ref[h, rows] = num
                        den_ref[h, rows] = den
                    else:
                        acc_ref[h, rows] = acc_ref[h, rows] + num
                        den_ref[h, rows] = den_ref[h, rows] + den
                return carry

            lax.fori_loop(0, HQ, head_step, 0)

        def remote_copy(src, dst, sem_idx, target):
            return pltpu.make_async_remote_copy(
                src_ref=src, dst_ref=dst,
                send_sem=send.at[sem_idx], recv_sem=recv.at[sem_idx],
                device_id=(target,), device_id_type=pl.DeviceIdType.MESH)

        top = pl.ds(0, HALF)
        bot = pl.ds(HALF, HALF)

        p1r_t = remote_copy(kv_ref.at[:, :, top], kvcom.at[0, :, :, top],
                            0, right)
        p1r_b = remote_copy(kv_ref.at[:, :, bot], kvcom.at[0, :, :, bot],
                            1, right)
        p1l_b = remote_copy(kv_ref.at[:, :, bot], kvcom.at[1, :, :, bot],
                            2, left)
        p1l_t = remote_copy(kv_ref.at[:, :, top], kvcom.at[1, :, :, top],
                            3, left)
        p1r_t.start()
        p1l_b.start()
        p1r_b.start()
        p1l_t.start()

        accumulate(kv_ref, True, 0, SEQ)

        p1r_t.wait_recv()
        p1l_b.wait_recv()

        p2r = remote_copy(kvcom.at[0, :, :, top], kvcom.at[2, :, :, top],
                          4, right)
        p2l = remote_copy(kvcom.at[1, :, :, bot], kvcom.at[2, :, :, bot],
                          5, left)
        p2r.start()
        p2l.start()

        accumulate(kvcom.at[0], False, 0, HALF)
        accumulate(kvcom.at[1], False, HALF, HALF)

        p1r_b.wait_recv()
        p1l_t.wait_recv()

        accumulate(kvcom.at[0], False, HALF, HALF)
        accumulate(kvcom.at[1], False, 0, HALF)

        p2r.wait_recv()
        p2l.wait_recv()

        accumulate(kvcom.at[2], False, 0, SEQ)

        for r in (p1r_t, p1r_b, p1l_b, p1l_t, p2r, p2l):
            r.wait_send()

        out_ref[...] = jnp.zeros((SEQ, D), jnp.float32)

        def proj_step(h, carry):
            ctx_h = acc_ref[h] / den_ref[h]
            out_ref[...] = out_ref[...] + jnp.dot(
                ctx_h, wo_ref[h], preferred_element_type=jnp.float32)
            return carry

        lax.fori_loop(0, HQ, proj_step, 0)

    out2d = pl.pallas_call(
        body,
        out_shape=jax.ShapeDtypeStruct((SEQ, D), jnp.float32),
        in_specs=[pl.BlockSpec(memory_space=pltpu.VMEM)] * 3,
        out_specs=pl.BlockSpec(memory_space=pltpu.VMEM),
        scratch_shapes=[
            pltpu.VMEM((3, 2, HQ, SEQ, DH), jnp.bfloat16),
            pltpu.VMEM((HQ, SEQ, DH), jnp.float32),
            pltpu.VMEM((HQ, SEQ, 1), jnp.float32),
            pltpu.SemaphoreType.DMA((6,)),
            pltpu.SemaphoreType.DMA((6,)),
        ],
        compiler_params=pltpu.CompilerParams(
            collective_id=0, vmem_limit_bytes=100 * 1024 * 1024),
    )(q, kv, wo)

    return out2d.reshape(1, SEQ, D)


# device time: 145657 ns/iter; 1.5291x vs baseline; 1.0378x over previous
import numpy as np
import jax
import jax.numpy as jnp
from jax import lax
from jax.experimental import pallas as pl
from jax.experimental.pallas import tpu as pltpu

N_DEV = 4
SEQ = 1024
D = 1024
HQ = 8
DH = 128
QT = 512
HALF = SEQ // 2
SCALE = 0.08838834764831843

_INV = (1.0 / (10000.0 ** (np.arange(0, DH, 2) / DH))).astype(np.float32)

_P = np.zeros((DH, DH), dtype=np.float32)
for _i in range(DH // 2):
    _P[2 * _i + 1, 2 * _i] = -1.0
    _P[2 * _i, 2 * _i + 1] = 1.0


def kernel(x, Wq, Wk, Wv, Wo):
    xs = x.reshape(SEQ, D).astype(jnp.bfloat16)

    my = lax.axis_index("i")
    pos = my * SEQ + jnp.arange(SEQ, dtype=jnp.float32)
    ang = pos[:, None] * jnp.asarray(_INV)[None, :]
    cos = jnp.repeat(jnp.cos(ang), 2, axis=-1)
    sin = jnp.repeat(jnp.sin(ang), 2, axis=-1)
    P = jnp.asarray(_P)

    wqkv = jnp.stack(
        [w.reshape(D, HQ, DH).transpose(1, 0, 2)
         for w in (Wq * SCALE, Wk, Wv)]).astype(jnp.bfloat16)
    wo = Wo.reshape(HQ, DH, D).astype(jnp.bfloat16)

    def body(x_ref, wqkv_ref, wo_ref, cos_ref, sin_ref, p_ref, out_ref,
             kvcom, q_scr, acc_ref, den_ref, send, recv):
        my_pos = lax.axis_index("i")
        left = lax.rem(my_pos + N_DEV - 1, N_DEV)
        right = lax.rem(my_pos + 1, N_DEV)

        barrier_sem = pltpu.get_barrier_semaphore()
        pl.semaphore_signal(barrier_sem, inc=1, device_id=(left,),
                            device_id_type=pl.DeviceIdType.MESH)
        pl.semaphore_signal(barrier_sem, inc=1, device_id=(right,),
                            device_id_type=pl.DeviceIdType.MESH)
        pl.semaphore_wait(barrier_sem, 2)

        cosv = cos_ref[...]
        sinv = sin_ref[...]
        pm = p_ref[...]

        def rot(t):
            return t * cosv + jnp.dot(
                t, pm, preferred_element_type=jnp.float32) * sinv

        def kv_step(h, carry):
            kh = jnp.dot(x_ref[...], wqkv_ref[1, h],
                         preferred_element_type=jnp.float32)
            kvcom[3, 0, h] = rot(kh).astype(jnp.bfloat16)
            vh = jnp.dot(x_ref[...], wqkv_ref[2, h],
                         preferred_element_type=jnp.float32)
            kvcom[3, 1, h] = vh.astype(jnp.bfloat16)
            return carry

        lax.fori_loop(0, HQ, kv_step, 0)

        def remote_copy(src, dst, sem_idx, target):
            return pltpu.make_async_remote_copy(
                src_ref=src, dst_ref=dst,
                send_sem=send.at[sem_idx], recv_sem=recv.at[sem_idx],
                device_id=(target,), device_id_type=pl.DeviceIdType.MESH)

        top = pl.ds(0, HALF)
        bot = pl.ds(HALF, HALF)

        p1r_t = remote_copy(kvcom.at[3, :, :, top], kvcom.at[0, :, :, top],
                            0, right)
        p1r_b = remote_copy(kvcom.at[3, :, :, bot], kvcom.at[0, :, :, bot],
                            1, right)
        p1l_b = remote_copy(kvcom.at[3, :, :, bot], kvcom.at[1, :, :, bot],
                            2, left)
        p1l_t = remote_copy(kvcom.at[3, :, :, top], kvcom.at[1, :, :, top],
                            3, left)
        p1r_t.start()
        p1l_b.start()
        p1r_b.start()
        p1l_t.start()

        def q_step(h, carry):
            qh = jnp.dot(x_ref[...], wqkv_ref[0, h],
                         preferred_element_type=jnp.float32)
            q_scr[h] = rot(qh).astype(jnp.bfloat16)
            return carry

        lax.fori_loop(0, HQ, q_step, 0)

        def accumulate(src, init, koff, klen):
            def head_step(h, carry):
                kc = src[0, h, pl.ds(koff, klen)]
                vc = src[1, h, pl.ds(koff, klen)]
                for c in range(SEQ // QT):
                    rows = pl.ds(c * QT, QT)
                    e = jnp.exp(lax.dot_general(
                        q_scr[h, rows], kc, (((1,), (1,)), ((), ())),
                        preferred_element_type=jnp.float32))
                    num = jnp.dot(e.astype(jnp.bfloat16), vc,
                                  preferred_element_type=jnp.float32)
                    den = jnp.sum(e, axis=-1, keepdims=True)
                    if init:
                        acc_ref[h, rows] = num
                        den_ref[h, rows] = den
                    else:
                        acc_ref[h, rows] = acc_ref[h, rows] + num
                        den_ref[h, rows] = den_ref[h, rows] + den
                return carry

            lax.fori_loop(0, HQ, head_step, 0)

        accumulate(kvcom.at[3], True, 0, SEQ)

        p1r_t.wait_recv()
        p1l_b.wait_recv()

        p2r = remote_copy(kvcom.at[0, :, :, top], kvcom.at[2, :, :, top],
                          4, right)
        p2l = remote_copy(kvcom.at[1, :, :, bot], kvcom.at[2, :, :, bot],
                          5, left)
        p2r.start()
        p2l.start()

        accumulate(kvcom.at[0], False, 0, HALF)
        accumulate(kvcom.at[1], False, HALF, HALF)

        p1r_b.wait_recv()
        p1l_t.wait_recv()

        accumulate(kvcom.at[0], False, HALF, HALF)
        accumulate(kvcom.at[1], False, 0, HALF)

        p2r.wait_recv()
        p2l.wait_recv()

        accumulate(kvcom.at[2], False, 0, SEQ)

        for r in (p1r_t, p1r_b, p1l_b, p1l_t, p2r, p2l):
            r.wait_send()

        out_ref[...] = jnp.zeros((SEQ, D), jnp.float32)

        def proj_step(h, carry):
            ctx_h = (acc_ref[h] / den_ref[h]).astype(jnp.bfloat16)
            out_ref[...] = out_ref[...] + jnp.dot(
                ctx_h, wo_ref[h], preferred_element_type=jnp.float32)
            return carry

        lax.fori_loop(0, HQ, proj_step, 0)

    out2d = pl.pallas_call(
        body,
        out_shape=jax.ShapeDtypeStruct((SEQ, D), jnp.float32),
        in_specs=[pl.BlockSpec(memory_space=pltpu.VMEM)] * 6,
        out_specs=pl.BlockSpec(memory_space=pltpu.VMEM),
        scratch_shapes=[
            pltpu.VMEM((4, 2, HQ, SEQ, DH), jnp.bfloat16),
            pltpu.VMEM((HQ, SEQ, DH), jnp.bfloat16),
            pltpu.VMEM((HQ, SEQ, DH), jnp.float32),
            pltpu.VMEM((HQ, SEQ, 1), jnp.float32),
            pltpu.SemaphoreType.DMA((6,)),
            pltpu.SemaphoreType.DMA((6,)),
        ],
        compiler_params=pltpu.CompilerParams(
            collective_id=0, vmem_limit_bytes=100 * 1024 * 1024),
    )(xs, wqkv, wo, cos, sin, P)

    return out2d.reshape(1, SEQ, D)


# device time: 138003 ns/iter; 1.6139x vs baseline; 1.0555x over previous
import numpy as np
import jax
import jax.numpy as jnp
from jax import lax
from jax.experimental import pallas as pl
from jax.experimental.pallas import tpu as pltpu

N_DEV = 4
SEQ = 1024
D = 1024
HQ = 8
DH = 128
QT = 512
HALF = SEQ // 2
SCALE = 0.08838834764831843

_INV = (1.0 / (10000.0 ** (np.arange(0, DH, 2) / DH))).astype(np.float32)

_P = np.zeros((DH, DH), dtype=np.float32)
for _i in range(DH // 2):
    _P[2 * _i + 1, 2 * _i] = -1.0
    _P[2 * _i, 2 * _i + 1] = 1.0


def kernel(x, Wq, Wk, Wv, Wo):
    xs = x.reshape(SEQ, D).astype(jnp.bfloat16)

    my = lax.axis_index("i")
    pos = my * SEQ + jnp.arange(SEQ, dtype=jnp.float32)
    ang = pos[:, None] * jnp.asarray(_INV)[None, :]
    cos = jnp.repeat(jnp.cos(ang), 2, axis=-1)
    sin = jnp.repeat(jnp.sin(ang), 2, axis=-1)
    P = jnp.asarray(_P)

    wqkv = jnp.stack(
        [w.reshape(D, HQ, DH).transpose(1, 0, 2)
         for w in (Wq * SCALE, Wk, Wv)]).astype(jnp.bfloat16)
    wo = Wo.reshape(HQ, DH, D).astype(jnp.bfloat16)

    def body(x_ref, wqkv_ref, wo_ref, cos_ref, sin_ref, p_ref, out_ref,
             kvcom, q_scr, acc_ref, den_ref, send, recv):
        my_pos = lax.axis_index("i")
        left = lax.rem(my_pos + N_DEV - 1, N_DEV)
        right = lax.rem(my_pos + 1, N_DEV)

        barrier_sem = pltpu.get_barrier_semaphore()
        pl.semaphore_signal(barrier_sem, inc=1, device_id=(left,),
                            device_id_type=pl.DeviceIdType.MESH)
        pl.semaphore_signal(barrier_sem, inc=1, device_id=(right,),
                            device_id_type=pl.DeviceIdType.MESH)
        pl.semaphore_wait(barrier_sem, 2)

        cosv = cos_ref[...]
        sinv = sin_ref[...]
        pm = p_ref[...]

        def rot(t):
            return t * cosv + jnp.dot(
                t, pm, preferred_element_type=jnp.float32) * sinv

        def kv_rows(roff):
            rows = slice(roff, roff + HALF)

            def kv_step(h, carry):
                xr = x_ref[rows]
                kh = jnp.dot(xr, wqkv_ref[1, h],
                             preferred_element_type=jnp.float32)
                kvcom[3, 0, h, rows] = (
                    kh * cosv[rows] + jnp.dot(
                        kh, pm, preferred_element_type=jnp.float32)
                    * sinv[rows]).astype(jnp.bfloat16)
                vh = jnp.dot(xr, wqkv_ref[2, h],
                             preferred_element_type=jnp.float32)
                kvcom[3, 1, h, rows] = vh.astype(jnp.bfloat16)
                return carry

            lax.fori_loop(0, HQ, kv_step, 0)

        def remote_copy(src, dst, sem_idx, target):
            return pltpu.make_async_remote_copy(
                src_ref=src, dst_ref=dst,
                send_sem=send.at[sem_idx], recv_sem=recv.at[sem_idx],
                device_id=(target,), device_id_type=pl.DeviceIdType.MESH)

        top = pl.ds(0, HALF)
        bot = pl.ds(HALF, HALF)

        p1r_t = remote_copy(kvcom.at[3, :, :, top], kvcom.at[0, :, :, top],
                            0, right)
        p1r_b = remote_copy(kvcom.at[3, :, :, bot], kvcom.at[0, :, :, bot],
                            1, right)
        p1l_b = remote_copy(kvcom.at[3, :, :, bot], kvcom.at[1, :, :, bot],
                            2, left)
        p1l_t = remote_copy(kvcom.at[3, :, :, top], kvcom.at[1, :, :, top],
                            3, left)
        kv_rows(0)
        p1r_t.start()
        p1l_t.start()
        kv_rows(HALF)
        p1r_b.start()
        p1l_b.start()

        def q_step(h, carry):
            qh = jnp.dot(x_ref[...], wqkv_ref[0, h],
                         preferred_element_type=jnp.float32)
            q_scr[h] = rot(qh).astype(jnp.bfloat16)
            return carry

        lax.fori_loop(0, HQ, q_step, 0)

        def accumulate(src, init, koff, klen):
            def head_step(h, carry):
                kc = src[0, h, pl.ds(koff, klen)]
                vc = src[1, h, pl.ds(koff, klen)]
                for c in range(SEQ // QT):
                    rows = pl.ds(c * QT, QT)
                    e = jnp.exp(lax.dot_general(
                        q_scr[h, rows], kc, (((1,), (1,)), ((), ())),
                        preferred_element_type=jnp.float32))
                    num = jnp.dot(e.astype(jnp.bfloat16), vc,
                                  preferred_element_type=jnp.float32)
                    den = jnp.sum(e, axis=-1, keepdims=True)
                    if init:
                        acc_ref[h, rows] = num
                        den_ref[h, rows] = den
                    else:
                        acc_ref[h, rows] = acc_ref[h, rows] + num
                        den_ref[h, rows] = den_ref[h, rows] + den
                return carry

            lax.fori_loop(0, HQ, head_step, 0)

        accumulate(kvcom.at[3], True, 0, SEQ)

        p1r_t.wait_recv()
        p2r = remote_copy(kvcom.at[0, :, :, top], kvcom.at[2, :, :, top],
                          4, right)
        p2r.start()
        p1l_b.wait_recv()
        p2l = remote_copy(kvcom.at[1, :, :, bot], kvcom.at[2, :, :, bot],
                          5, left)
        p2l.start()

        p1r_b.wait_recv()
        p1l_t.wait_recv()

        accumulate(kvcom.at[0], False, 0, SEQ)
        accumulate(kvcom.at[1], False, 0, SEQ)

        p2r.wait_recv()
        p2l.wait_recv()

        accumulate(kvcom.at[2], False, 0, SEQ)

        for r in (p1r_t, p1r_b, p1l_b, p1l_t, p2r, p2l):
            r.wait_send()

        out_ref[...] = jnp.zeros((SEQ, D), jnp.float32)

        def proj_step(h, carry):
            ctx_h = (acc_ref[h] / den_ref[h]).astype(jnp.bfloat16)
            out_ref[...] = out_ref[...] + jnp.dot(
                ctx_h, wo_ref[h], preferred_element_type=jnp.float32)
            return carry

        lax.fori_loop(0, HQ, proj_step, 0)

    out2d = pl.pallas_call(
        body,
        out_shape=jax.ShapeDtypeStruct((SEQ, D), jnp.float32),
        in_specs=[pl.BlockSpec(memory_space=pltpu.VMEM)] * 6,
        out_specs=pl.BlockSpec(memory_space=pltpu.VMEM),
        scratch_shapes=[
            pltpu.VMEM((4, 2, HQ, SEQ, DH), jnp.bfloat16),
            pltpu.VMEM((HQ, SEQ, DH), jnp.bfloat16),
            pltpu.VMEM((HQ, SEQ, DH), jnp.float32),
            pltpu.VMEM((HQ, SEQ, 1), jnp.float32),
            pltpu.SemaphoreType.DMA((6,)),
            pltpu.SemaphoreType.DMA((6,)),
        ],
        compiler_params=pltpu.CompilerParams(
            collective_id=0, vmem_limit_bytes=100 * 1024 * 1024),
    )(xs, wqkv, wo, cos, sin, P)

    return out2d.reshape(1, SEQ, D)


# device time: 137939 ns/iter; 1.6147x vs baseline; 1.0005x over previous
import numpy as np
import jax
import jax.numpy as jnp
from jax import lax
from jax.experimental import pallas as pl
from jax.experimental.pallas import tpu as pltpu

N_DEV = 4
SEQ = 1024
D = 1024
HQ = 8
DH = 128
QT = 1024
HALF = SEQ // 2
SCALE = 0.08838834764831843

_INV = (1.0 / (10000.0 ** (np.arange(0, DH, 2) / DH))).astype(np.float32)

_P = np.zeros((DH, DH), dtype=np.float32)
for _i in range(DH // 2):
    _P[2 * _i + 1, 2 * _i] = -1.0
    _P[2 * _i, 2 * _i + 1] = 1.0


def kernel(x, Wq, Wk, Wv, Wo):
    xs = x.reshape(SEQ, D).astype(jnp.bfloat16)

    my = lax.axis_index("i")
    pos = my * SEQ + jnp.arange(SEQ, dtype=jnp.float32)
    ang = pos[:, None] * jnp.asarray(_INV)[None, :]
    cos = jnp.repeat(jnp.cos(ang), 2, axis=-1)
    sin = jnp.repeat(jnp.sin(ang), 2, axis=-1)
    P = jnp.asarray(_P)

    wqkv = jnp.stack(
        [w.reshape(D, HQ, DH).transpose(1, 0, 2)
         for w in (Wq * SCALE, Wk, Wv)]).astype(jnp.bfloat16)
    wo = Wo.reshape(HQ, DH, D).astype(jnp.bfloat16)

    def body(x_ref, wqkv_ref, wo_ref, cos_ref, sin_ref, p_ref, out_ref,
             kvcom, q_scr, acc_ref, den_ref, send, recv):
        my_pos = lax.axis_index("i")
        left = lax.rem(my_pos + N_DEV - 1, N_DEV)
        right = lax.rem(my_pos + 1, N_DEV)

        barrier_sem = pltpu.get_barrier_semaphore()
        pl.semaphore_signal(barrier_sem, inc=1, device_id=(left,),
                            device_id_type=pl.DeviceIdType.MESH)
        pl.semaphore_signal(barrier_sem, inc=1, device_id=(right,),
                            device_id_type=pl.DeviceIdType.MESH)
        pl.semaphore_wait(barrier_sem, 2)

        cosv = cos_ref[...]
        sinv = sin_ref[...]
        pm = p_ref[...]

        def rot(t):
            return t * cosv + jnp.dot(
                t, pm, preferred_element_type=jnp.float32) * sinv

        def kv_rows(roff):
            rows = slice(roff, roff + HALF)

            def kv_step(h, carry):
                xr = x_ref[rows]
                kh = jnp.dot(xr, wqkv_ref[1, h],
                             preferred_element_type=jnp.float32)
                kvcom[3, 0, h, rows] = (
                    kh * cosv[rows] + jnp.dot(
                        kh, pm, preferred_element_type=jnp.float32)
                    * sinv[rows]).astype(jnp.bfloat16)
                vh = jnp.dot(xr, wqkv_ref[2, h],
                             preferred_element_type=jnp.float32)
                kvcom[3, 1, h, rows] = vh.astype(jnp.bfloat16)
                return carry

            lax.fori_loop(0, HQ, kv_step, 0)

        def remote_copy(src, dst, sem_idx, target):
            return pltpu.make_async_remote_copy(
                src_ref=src, dst_ref=dst,
                send_sem=send.at[sem_idx], recv_sem=recv.at[sem_idx],
                device_id=(target,), device_id_type=pl.DeviceIdType.MESH)

        top = pl.ds(0, HALF)
        bot = pl.ds(HALF, HALF)

        p1r_t = remote_copy(kvcom.at[3, :, :, top], kvcom.at[0, :, :, top],
                            0, right)
        p1r_b = remote_copy(kvcom.at[3, :, :, bot], kvcom.at[0, :, :, bot],
                            1, right)
        p1l_b = remote_copy(kvcom.at[3, :, :, bot], kvcom.at[1, :, :, bot],
                            2, left)
        p1l_t = remote_copy(kvcom.at[3, :, :, top], kvcom.at[1, :, :, top],
                            3, left)
        kv_rows(0)
        p1r_t.start()
        p1l_t.start()
        kv_rows(HALF)
        p1r_b.start()
        p1l_b.start()

        def q_step(h, carry):
            qh = jnp.dot(x_ref[...], wqkv_ref[0, h],
                         preferred_element_type=jnp.float32)
            q_scr[h] = rot(qh).astype(jnp.bfloat16)
            return carry

        lax.fori_loop(0, HQ, q_step, 0)

        def accumulate(src, init, koff, klen):
            def head_step(h, carry):
                kc = src[0, h, pl.ds(koff, klen)]
                vc = src[1, h, pl.ds(koff, klen)]
                for c in range(SEQ // QT):
                    rows = pl.ds(c * QT, QT)
                    e = jnp.exp(lax.dot_general(
                        q_scr[h, rows], kc, (((1,), (1,)), ((), ())),
                        preferred_element_type=jnp.float32))
                    num = jnp.dot(e.astype(jnp.bfloat16), vc,
                                  preferred_element_type=jnp.float32)
                    den = jnp.sum(e, axis=-1, keepdims=True)
                    if init:
                        acc_ref[h, rows] = num
                        den_ref[h, rows] = den
                    else:
                        acc_ref[h, rows] = acc_ref[h, rows] + num
                        den_ref[h, rows] = den_ref[h, rows] + den
                return carry

            lax.fori_loop(0, HQ, head_step, 0)

        accumulate(kvcom.at[3], True, 0, SEQ)

        p1r_t.wait_recv()
        p2r = remote_copy(kvcom.at[0, :, :, top], kvcom.at[2, :, :, top],
                          4, right)
        p2r.start()
        p1l_b.wait_recv()
        p2l = remote_copy(kvcom.at[1, :, :, bot], kvcom.at[2, :, :, bot],
                          5, left)
        p2l.start()

        p1r_b.wait_recv()
        p1l_t.wait_recv()

        accumulate(kvcom.at[0], False, 0, SEQ)
        accumulate(kvcom.at[1], False, 0, SEQ)

        p2r.wait_recv()
        p2l.wait_recv()

        accumulate(kvcom.at[2], False, 0, SEQ)

        for r in (p1r_t, p1r_b, p1l_b, p1l_t, p2r, p2l):
            r.wait_send()

        out_ref[...] = jnp.zeros((SEQ, D), jnp.float32)

        def proj_step(h, carry):
            ctx_h = (acc_ref[h] / den_ref[h]).astype(jnp.bfloat16)
            out_ref[...] = out_ref[...] + jnp.dot(
                ctx_h, wo_ref[h], preferred_element_type=jnp.float32)
            return carry

        lax.fori_loop(0, HQ, proj_step, 0)

    out2d = pl.pallas_call(
        body,
        out_shape=jax.ShapeDtypeStruct((SEQ, D), jnp.float32),
        in_specs=[pl.BlockSpec(memory_space=pltpu.VMEM)] * 6,
        out_specs=pl.BlockSpec(memory_space=pltpu.VMEM),
        scratch_shapes=[
            pltpu.VMEM((4, 2, HQ, SEQ, DH), jnp.bfloat16),
            pltpu.VMEM((HQ, SEQ, DH), jnp.bfloat16),
            pltpu.VMEM((HQ, SEQ, DH), jnp.float32),
            pltpu.VMEM((HQ, SEQ, 1), jnp.float32),
            pltpu.SemaphoreType.DMA((6,)),
            pltpu.SemaphoreType.DMA((6,)),
        ],
        compiler_params=pltpu.CompilerParams(
            collective_id=0, vmem_limit_bytes=100 * 1024 * 1024),
    )(xs, wqkv, wo, cos, sin, P)

    return out2d.reshape(1, SEQ, D)
